# baseline (speedup 1.0000x reference)
import functools
import jax
import jax.numpy as jnp
from jax import lax
from jax.experimental import pallas as pl
from jax.experimental.pallas import tpu as pltpu
from jax.experimental.pallas import tpu_sc as plsc

N_B = 64
N_S = 576
N_ROWS = N_B * N_S
E_DIM = 64
N_CODES = 1024

NW = 32
B_PER_W = N_ROWS // NW
CHUNK = 128
N_CHUNKS = B_PER_W // CHUNK


B_BLK = 32


def _argmax_kernel(xt_ref, embed_ref, idx_ref, scaled_ref):
    @pl.when(pl.program_id(0) == 0)
    def _():
        emb = embed_ref[...]
        inv_sq = 1.0 / jnp.sum(emb * emb, axis=1, keepdims=True)
        scaled_ref[...] = emb * inv_sq

    scaled = scaled_ref[...]
    for b in range(B_BLK):
        xb = xt_ref[b]
        simT = jax.lax.dot_general(
            scaled, xb, (((1,), (0,)), ((), ())),
            preferred_element_type=jnp.float32)
        idx_ref[b] = jnp.argmax(simT, axis=0).astype(jnp.int32)[None, :]


def _tc_argmax(xt, embed):
    return pl.pallas_call(
        _argmax_kernel,
        grid=(N_B // B_BLK,),
        in_specs=[
            pl.BlockSpec((B_BLK, E_DIM, N_S), lambda i: (i, 0, 0)),
            pl.BlockSpec((N_CODES, E_DIM), lambda i: (0, 0)),
        ],
        out_specs=pl.BlockSpec((B_BLK, 1, N_S), lambda i: (i, 0, 0)),
        out_shape=jax.ShapeDtypeStruct((N_B, 1, N_S), jnp.int32),
        scratch_shapes=[pltpu.VMEM((N_CODES, E_DIM), jnp.float32)],
    )(xt, embed)


_sc_mesh = plsc.VectorSubcoreMesh(core_axis_name="c", subcore_axis_name="s")


@functools.partial(
    pl.kernel,
    mesh=_sc_mesh,
    compiler_params=pltpu.CompilerParams(use_tc_tiling_on_sc=False),
    out_type=jax.ShapeDtypeStruct((N_ROWS, E_DIM), jnp.float32),
    scratch_types=[
        pltpu.VMEM((B_PER_W,), jnp.int32),
        pltpu.VMEM((B_PER_W, E_DIM), jnp.float32),
        pltpu.SemaphoreType.DMA,
    ],
)
def _sc_gather(idx_hbm, embed_hbm, out_hbm, idx_v, rows_v, sem):
    wid = lax.axis_index("s") * 2 + lax.axis_index("c")
    base = wid * B_PER_W
    pltpu.sync_copy(idx_hbm.at[pl.ds(base, B_PER_W)], idx_v)
    copies = []
    for j in range(N_CHUNKS):
        copies.append(pltpu.async_copy(
            embed_hbm.at[idx_v.at[pl.ds(j * CHUNK, CHUNK)]],
            rows_v.at[pl.ds(j * CHUNK, CHUNK)],
            sem))
    for c in copies:
        c.wait()
    pltpu.sync_copy(rows_v, out_hbm.at[pl.ds(base, B_PER_W)])


def kernel(x, embed):
    xt = jnp.transpose(x, (0, 2, 1))
    idx2d = _tc_argmax(xt, embed)
    idx = idx2d.reshape(N_ROWS)
    quant = _sc_gather(idx, embed)
    return quant, idx


# speedup vs baseline: 1.0049x; 1.0049x over previous
import functools
import jax
import jax.numpy as jnp
from jax import lax
from jax.experimental import pallas as pl
from jax.experimental.pallas import tpu as pltpu
from jax.experimental.pallas import tpu_sc as plsc

N_B = 64
N_S = 576
N_ROWS = N_B * N_S
E_DIM = 64
N_CODES = 1024

NW = 32
B_PER_W = N_ROWS // NW
CHUNK = 128
N_CHUNKS = B_PER_W // CHUNK


B_BLK = 16


def _argmax_kernel(xt_ref, embed_ref, idx_ref, scaled_ref):
    @pl.when(pl.program_id(0) == 0)
    def _():
        emb = embed_ref[...]
        norm = jnp.sqrt(jnp.sum(emb * emb, axis=1, keepdims=True))
        scaled_ref[...] = emb / (norm * norm)

    scaled = scaled_ref[...]
    for b in range(B_BLK):
        xb = xt_ref[b]
        simT = jax.lax.dot_general(
            scaled, xb, (((1,), (0,)), ((), ())),
            preferred_element_type=jnp.float32)
        idx_ref[b] = jnp.argmax(simT, axis=0).astype(jnp.int32)[None, :]


def _tc_argmax(xt, embed):
    return pl.pallas_call(
        _argmax_kernel,
        grid=(N_B // B_BLK,),
        in_specs=[
            pl.BlockSpec((B_BLK, E_DIM, N_S), lambda i: (i, 0, 0)),
            pl.BlockSpec((N_CODES, E_DIM), lambda i: (0, 0)),
        ],
        out_specs=pl.BlockSpec((B_BLK, 1, N_S), lambda i: (i, 0, 0)),
        out_shape=jax.ShapeDtypeStruct((N_B, 1, N_S), jnp.int32),
        scratch_shapes=[pltpu.VMEM((N_CODES, E_DIM), jnp.float32)],
    )(xt, embed)


_sc_mesh = plsc.VectorSubcoreMesh(core_axis_name="c", subcore_axis_name="s")


@functools.partial(
    pl.kernel,
    mesh=_sc_mesh,
    compiler_params=pltpu.CompilerParams(use_tc_tiling_on_sc=False),
    out_type=jax.ShapeDtypeStruct((N_ROWS, E_DIM), jnp.float32),
    scratch_types=[
        pltpu.VMEM((B_PER_W,), jnp.int32),
        pltpu.VMEM((B_PER_W, E_DIM), jnp.float32),
        pltpu.SemaphoreType.DMA,
    ],
)
def _sc_gather(idx_hbm, embed_hbm, out_hbm, idx_v, rows_v, sem):
    wid = lax.axis_index("s") * 2 + lax.axis_index("c")
    base = wid * B_PER_W
    pltpu.sync_copy(idx_hbm.at[pl.ds(base, B_PER_W)], idx_v)
    copies = []
    for j in range(N_CHUNKS):
        copies.append(pltpu.async_copy(
            embed_hbm.at[idx_v.at[pl.ds(j * CHUNK, CHUNK)]],
            rows_v.at[pl.ds(j * CHUNK, CHUNK)],
            sem))
    for c in copies:
        c.wait()
    pltpu.sync_copy(rows_v, out_hbm.at[pl.ds(base, B_PER_W)])


def kernel(x, embed):
    xt = jnp.transpose(x, (0, 2, 1))
    idx2d = _tc_argmax(xt, embed)
    idx = idx2d.reshape(N_ROWS)
    quant = _sc_gather(idx, embed)
    return quant, idx


# speedup vs baseline: 1.0270x; 1.0220x over previous
import functools
import jax
import jax.numpy as jnp
from jax import lax
from jax.experimental import pallas as pl
from jax.experimental.pallas import tpu as pltpu
from jax.experimental.pallas import tpu_sc as plsc

N_B = 64
N_S = 576
N_ROWS = N_B * N_S
E_DIM = 64
N_CODES = 1024

NW = 32
B_PER_W = N_ROWS // NW
CHUNK = 128
N_CHUNKS = B_PER_W // CHUNK


B_BLK = 16


def _argmax_kernel(xt_ref, embed_ref, idx_ref, scaled_ref):
    @pl.when(pl.program_id(0) == 0)
    def _():
        emb = embed_ref[...]
        norm = jnp.sqrt(jnp.sum(emb * emb, axis=1, keepdims=True))
        scaled_ref[...] = emb / (norm * norm)

    scaled = scaled_ref[...]
    for b in range(B_BLK):
        xb = xt_ref[b]
        simT = jax.lax.dot_general(
            scaled, xb, (((1,), (0,)), ((), ())),
            preferred_element_type=jnp.float32)
        idx_ref[pl.ds(b * N_S, N_S)] = jnp.argmax(simT, axis=0).astype(jnp.int32)


def _tc_argmax(xt, embed):
    return pl.pallas_call(
        _argmax_kernel,
        grid=(N_B // B_BLK,),
        in_specs=[
            pl.BlockSpec((B_BLK, E_DIM, N_S), lambda i: (i, 0, 0)),
            pl.BlockSpec((N_CODES, E_DIM), lambda i: (0, 0)),
        ],
        out_specs=pl.BlockSpec((B_BLK * N_S,), lambda i: (i,)),
        out_shape=jax.ShapeDtypeStruct((N_ROWS,), jnp.int32),
        scratch_shapes=[pltpu.VMEM((N_CODES, E_DIM), jnp.float32)],
    )(xt, embed)


_sc_mesh = plsc.VectorSubcoreMesh(core_axis_name="c", subcore_axis_name="s")


@functools.partial(
    pl.kernel,
    mesh=_sc_mesh,
    compiler_params=pltpu.CompilerParams(use_tc_tiling_on_sc=False),
    out_type=jax.ShapeDtypeStruct((N_ROWS, E_DIM), jnp.float32),
    scratch_types=[
        pltpu.VMEM((B_PER_W,), jnp.int32),
        pltpu.VMEM((B_PER_W, E_DIM), jnp.float32),
        pltpu.SemaphoreType.DMA,
    ],
)
def _sc_gather(idx_hbm, embed_hbm, out_hbm, idx_v, rows_v, sem):
    wid = lax.axis_index("s") * 2 + lax.axis_index("c")
    base = wid * B_PER_W
    pltpu.sync_copy(idx_hbm.at[pl.ds(base, B_PER_W)], idx_v)
    copies = []
    for j in range(N_CHUNKS):
        copies.append(pltpu.async_copy(
            embed_hbm.at[idx_v.at[pl.ds(j * CHUNK, CHUNK)]],
            rows_v.at[pl.ds(j * CHUNK, CHUNK)],
            sem))
    for c in copies:
        c.wait()
    pltpu.sync_copy(rows_v, out_hbm.at[pl.ds(base, B_PER_W)])


def kernel(x, embed):
    xt = jnp.transpose(x, (0, 2, 1))
    idx = _tc_argmax(xt, embed)
    quant = _sc_gather(idx, embed)
    return quant, idx
